# Initial kernel scaffold; baseline (speedup 1.0000x reference)
import jax
import jax.numpy as jnp
from jax.experimental import pallas as pl


def kernel(template_feature, search_feature, template_xyz, search_xyz, template_bc, search_bc, params):
    raise NotImplementedError("write your pallas kernel here")


# speedup vs baseline: 1.0912x; 1.0912x over previous
import jax
import jax.numpy as jnp
from jax.experimental import pallas as pl

_EPS = 1e-12
_K1, _K2, _KC, _RADIUS = 8, 4, 16, 0.5
_INTERPRET = False


def _pdist(a, b):
    return jnp.sqrt(jnp.sum((a[:, :, None, :] - b[:, None, :, :]) ** 2, axis=-1) + _EPS)


def _scf_oe_t(xyz):
    c = xyz - jnp.mean(xyz, axis=1, keepdims=True)
    r = jnp.sqrt(jnp.sum(c ** 2, axis=-1) + _EPS)
    theta = jnp.arccos(jnp.clip(c[..., 2] / r, -1.0 + 1e-6, 1.0 - 1e-6))
    phi = jnp.arctan2(c[..., 1], c[..., 0] + 1e-8)
    return jnp.stack([r, theta, phi], axis=-1)


def _scf_desc(xyz, kc):
    d = _pdist(xyz, xyz)
    vals = -jax.lax.top_k(-d, kc + 1)[0]
    return vals[:, :, 1:kc + 1]


def _inorm(x):
    m = jnp.mean(x, axis=1, keepdims=True)
    v = jnp.var(x, axis=1, keepdims=True)
    return (x - m) / jnp.sqrt(v + 1e-5)


def _attn0_body(x_ref, r_ref, wq_ref, wk_ref, wv_ref, pw1_ref, pb1_ref,
                pw2_ref, pb2_ref, aw1_ref, ab1_ref, aw2_ref, ab2_ref, o_ref):
    X = x_ref[...]
    R = r_ref[...]
    G5 = X.shape[0]
    G = G5 // 5
    pe = jax.nn.relu(jnp.dot(R, pw1_ref[...], preferred_element_type=jnp.float32)
                     + pb1_ref[...])
    pe = jnp.dot(pe, pw2_ref[...], preferred_element_type=jnp.float32) + pb2_ref[...]
    q = jnp.dot(X, wq_ref[...], preferred_element_type=jnp.float32)
    k = jnp.dot(X, wk_ref[...], preferred_element_type=jnp.float32)
    v = jnp.dot(X, wv_ref[...], preferred_element_type=jnp.float32)
    pe3 = pe.reshape(G, 5, 128)
    t = q.reshape(G, 5, 128)[:, 0:1, :] - k.reshape(G, 5, 128) + pe3
    h = jax.nn.relu(jnp.dot(t.reshape(G5, 128), aw1_ref[...],
                            preferred_element_type=jnp.float32) + ab1_ref[...])
    h = jnp.dot(h, aw2_ref[...], preferred_element_type=jnp.float32) + ab2_ref[...]
    h3 = h.reshape(G, 5, 128)
    m = jnp.max(h3, axis=1, keepdims=True)
    e = jnp.exp(h3 - m)
    a = e / jnp.sum(e, axis=1, keepdims=True)
    vv = v.reshape(G, 5, 128) + pe3
    o_ref[...] = jnp.sum(a * vv, axis=1)


def _attn0(x, rel0, p):
    Bn = x.shape[0]
    G = 512
    X2 = x.reshape(Bn * 5, 128)
    R2 = rel0.reshape(Bn * 5, 12)
    pb1 = p['pos_b1'].reshape(1, -1)
    pb2 = p['pos_b2'].reshape(1, -1)
    ab1 = p['att_b1'].reshape(1, -1)
    ab2 = p['att_b2'].reshape(1, -1)
    full = lambda a: pl.BlockSpec(a.shape, lambda i: (0, 0))
    return pl.pallas_call(
        _attn0_body,
        grid=(Bn // G,),
        in_specs=[
            pl.BlockSpec((G * 5, 128), lambda i: (i, 0)),
            pl.BlockSpec((G * 5, 12), lambda i: (i, 0)),
            full(p['wq']), full(p['wk']), full(p['wv']),
            full(p['pos_w1']), full(pb1), full(p['pos_w2']), full(pb2),
            full(p['att_w1']), full(ab1), full(p['att_w2']), full(ab2),
        ],
        out_specs=pl.BlockSpec((G, 128), lambda i: (i, 0)),
        out_shape=jax.ShapeDtypeStruct((Bn, 128), jnp.float32),
        interpret=_INTERPRET,
    )(X2, R2, p['wq'], p['wk'], p['wv'], p['pos_w1'], pb1, p['pos_w2'], pb2,
      p['att_w1'], ab1, p['att_w2'], ab2)


def _bn_nlast(y, g, b):
    red = tuple(range(y.ndim - 1))
    m = jnp.mean(y, axis=red, keepdims=True)
    v = jnp.var(y, axis=red, keepdims=True)
    return (y - m) / jnp.sqrt(v + 1e-5) * g + b


def _pointsift(xyz, feats, layers, radius):
    B, N, _ = xyz.shape
    rel = xyz[:, None, :, :] - xyz[:, :, None, :]
    dist = jnp.sqrt(jnp.sum(rel ** 2, axis=-1) + _EPS)
    oc = ((rel[..., 0] >= 0).astype(jnp.int32) * 4
          + (rel[..., 1] >= 0).astype(jnp.int32) * 2
          + (rel[..., 2] >= 0).astype(jnp.int32))
    eye = jnp.eye(N, dtype=bool)[None]
    idxs = []
    for o in range(8):
        valid = (oc == o) & (dist <= radius) & (~eye)
        md = jnp.where(valid, dist, jnp.inf)
        j = jnp.argmin(md, axis=-1)
        has = jnp.isfinite(jnp.min(md, axis=-1))
        idxs.append(jnp.where(has, j, jnp.arange(N)[None, :]))
    idx = jnp.stack(idxs, axis=-1)
    bi = jnp.arange(B)[:, None, None]
    gx = xyz[bi, idx] - xyz[:, :, None, :]
    gf = feats[bi, idx]
    x = jnp.concatenate([gx, gf], axis=-1)
    for (W, bb, g, b) in layers:
        Bc, Nn, T, C = x.shape
        xp = x.reshape(Bc, Nn, T // 2, 2, C)
        y = jnp.einsum('bntsc,sco->bnto', xp, W) + bb
        x = jax.nn.relu(_bn_nlast(y, g, b))
    return jnp.transpose(x[:, :, 0, :], (0, 2, 1))


def kernel(template_feature, search_feature, template_xyz, search_xyz,
           template_bc, search_bc, params):
    B, f, M = template_feature.shape
    N = search_feature.shape[2]

    d_bc = _pdist(search_bc, template_bc)
    _, idx_b = jax.lax.top_k(-d_bc, 3 * _K1)
    dxyz = _pdist(search_xyz, template_xyz)
    bi = jnp.arange(B)[:, None, None]
    ni = jnp.arange(N)[None, :, None]
    mask = jnp.full((B, N, M), 1e5, dtype=jnp.float32).at[bi, ni, idx_b].set(1.0)
    _, idx_k = jax.lax.top_k(-(dxyz * mask), _K1)

    s_desc = _scf_desc(search_xyz, _KC)
    t_desc = _scf_desc(template_xyz, _KC)
    cand = t_desc[bi, idx_k]
    dsc = jnp.sqrt(jnp.sum((s_desc[:, :, None, :] - cand) ** 2, axis=-1) + _EPS)
    _, pos4 = jax.lax.top_k(-dsc, _K2)
    idx_scf = jnp.take_along_axis(idx_k, pos4, axis=-1)

    t_oe = _scf_oe_t(template_xyz)
    s_oe = _scf_oe_t(search_xyz)
    t_feat = jnp.concatenate(
        [jnp.transpose(template_feature, (0, 2, 1)), t_oe], axis=-1)
    s_feat = jnp.concatenate(
        [jnp.transpose(search_feature, (0, 2, 1)), s_oe], axis=-1)
    g_feat = t_feat[bi, idx_scf]
    fr = jnp.concatenate([s_feat[:, :, None, :], g_feat], axis=2)

    g_xyz = template_xyz[bi, idx_scf]
    g_bc = template_bc[bi, idx_scf]
    st = jnp.concatenate([
        jnp.concatenate([search_xyz[:, :, None, :], g_xyz], axis=2),
        jnp.concatenate([search_bc[:, :, None, :], g_bc], axis=2),
    ], axis=-1)

    x = fr
    for (W, g, b) in params['mlp']:
        y = jnp.einsum('bnkc,oc->bnko', x, W)
        x = jax.nn.relu(_bn_nlast(y, g, b))

    Bn = B * N
    rel0 = st[:, :, 0:1, :] - st
    ff = _attn0(x.reshape(Bn, _K2 + 1, f),
                rel0.reshape(Bn, _K2 + 1, 12), params['attn'])
    ff = jnp.transpose(ff.reshape(B, N, f), (0, 2, 1))

    ff = _inorm(ff + search_feature)
    y = jnp.einsum('bcn,oc->bon', ff, params['fea_w1'])
    m = jnp.mean(y, axis=(0, 2), keepdims=True)
    v = jnp.var(y, axis=(0, 2), keepdims=True)
    y = jax.nn.relu((y - m) / jnp.sqrt(v + 1e-5)
                    * params['fea_g1'][None, :, None] + params['fea_b1'][None, :, None])
    y = jnp.einsum('bcn,oc->bon', y, params['fea_w2']) + params['fea_bias2'][None, :, None]
    fff = _inorm(y + ff)

    fc = _pointsift(search_xyz, jnp.transpose(fff, (0, 2, 1)), params['oe1'], _RADIUS)
    fc = fc + fff
    fcf = _pointsift(search_xyz, jnp.transpose(fc, (0, 2, 1)), params['oe2'], _RADIUS)
    return fcf + fc


# speedup vs baseline: 1.1084x; 1.0157x over previous
import jax
import jax.numpy as jnp
from jax.experimental import pallas as pl

_EPS = 1e-12
_K1, _K2, _KC, _RADIUS = 8, 4, 16, 0.5
_INTERPRET = False


def _pdist(a, b):
    return jnp.sqrt(jnp.sum((a[:, :, None, :] - b[:, None, :, :]) ** 2, axis=-1) + _EPS)


def _scf_oe_t(xyz):
    c = xyz - jnp.mean(xyz, axis=1, keepdims=True)
    r = jnp.sqrt(jnp.sum(c ** 2, axis=-1) + _EPS)
    theta = jnp.arccos(jnp.clip(c[..., 2] / r, -1.0 + 1e-6, 1.0 - 1e-6))
    phi = jnp.arctan2(c[..., 1], c[..., 0] + 1e-8)
    return jnp.stack([r, theta, phi], axis=-1)


def _scf_desc(xyz, kc):
    d = _pdist(xyz, xyz)
    vals = -jax.lax.top_k(-d, kc + 1)[0]
    return vals[:, :, 1:kc + 1]


def _inorm(x):
    m = jnp.mean(x, axis=1, keepdims=True)
    v = jnp.var(x, axis=1, keepdims=True)
    return (x - m) / jnp.sqrt(v + 1e-5)


def _attn0_body(x_ref, r_ref, wq_ref, wk_ref, wv_ref, pw1_ref, pb1_ref,
                pw2_ref, pb2_ref, aw1_ref, ab1_ref, aw2_ref, ab2_ref, o_ref):
    X = x_ref[...]
    R = r_ref[...]
    G5 = X.shape[0]
    G = G5 // 5
    pe = jax.nn.relu(jnp.dot(R, pw1_ref[...], preferred_element_type=jnp.float32)
                     + pb1_ref[...])
    pe = jnp.dot(pe, pw2_ref[...], preferred_element_type=jnp.float32) + pb2_ref[...]
    q = jnp.dot(X, wq_ref[...], preferred_element_type=jnp.float32)
    k = jnp.dot(X, wk_ref[...], preferred_element_type=jnp.float32)
    v = jnp.dot(X, wv_ref[...], preferred_element_type=jnp.float32)
    pe3 = pe.reshape(G, 5, 128)
    t = q.reshape(G, 5, 128)[:, 0:1, :] - k.reshape(G, 5, 128) + pe3
    h = jax.nn.relu(jnp.dot(t.reshape(G5, 128), aw1_ref[...],
                            preferred_element_type=jnp.float32) + ab1_ref[...])
    h = jnp.dot(h, aw2_ref[...], preferred_element_type=jnp.float32) + ab2_ref[...]
    h3 = h.reshape(G, 5, 128)
    m = jnp.max(h3, axis=1, keepdims=True)
    e = jnp.exp(h3 - m)
    a = e / jnp.sum(e, axis=1, keepdims=True)
    vv = v.reshape(G, 5, 128) + pe3
    o_ref[...] = jnp.sum(a * vv, axis=1)


def _attn0(x, rel0, p):
    Bn = x.shape[0]
    G = 512
    X2 = x.reshape(Bn * 5, 128)
    R2 = rel0.reshape(Bn * 5, 12)
    pb1 = p['pos_b1'].reshape(1, -1)
    pb2 = p['pos_b2'].reshape(1, -1)
    ab1 = p['att_b1'].reshape(1, -1)
    ab2 = p['att_b2'].reshape(1, -1)
    full = lambda a: pl.BlockSpec(a.shape, lambda i: (0, 0))
    return pl.pallas_call(
        _attn0_body,
        grid=(Bn // G,),
        in_specs=[
            pl.BlockSpec((G * 5, 128), lambda i: (i, 0)),
            pl.BlockSpec((G * 5, 12), lambda i: (i, 0)),
            full(p['wq']), full(p['wk']), full(p['wv']),
            full(p['pos_w1']), full(pb1), full(p['pos_w2']), full(pb2),
            full(p['att_w1']), full(ab1), full(p['att_w2']), full(ab2),
        ],
        out_specs=pl.BlockSpec((G, 128), lambda i: (i, 0)),
        out_shape=jax.ShapeDtypeStruct((Bn, 128), jnp.float32),
        interpret=_INTERPRET,
    )(X2, R2, p['wq'], p['wk'], p['wv'], p['pos_w1'], pb1, p['pos_w2'], pb2,
      p['att_w1'], ab1, p['att_w2'], ab2)


def _psift_idx_body(xq_ref, xa_ref, o_ref):
    G = xq_ref.shape[1]
    N = xa_ref.shape[2]
    i = pl.program_id(1)
    dx = xa_ref[0, 0:1, :] - xq_ref[0, :, 0:1]
    dy = xa_ref[0, 1:2, :] - xq_ref[0, :, 1:2]
    dz = xa_ref[0, 2:3, :] - xq_ref[0, :, 2:3]
    dist = jnp.sqrt(dx * dx + dy * dy + dz * dz + _EPS)
    oc = ((dx >= 0).astype(jnp.int32) * 4
          + (dy >= 0).astype(jnp.int32) * 2
          + (dz >= 0).astype(jnp.int32))
    jj = jax.lax.broadcasted_iota(jnp.int32, (G, N), 1)
    rows = jax.lax.broadcasted_iota(jnp.int32, (G, N), 0) + i * G
    eye = jj == rows
    self_idx = rows[:, 0]
    cols = []
    for o in range(8):
        valid = (oc == o) & (dist <= _RADIUS) & (~eye)
        md = jnp.where(valid, dist, 1e30)
        mn = jnp.min(md, axis=1)
        j = jnp.argmin(md, axis=1).astype(jnp.int32)
        cols.append(jnp.where(mn < 1e30, j, self_idx))
    o_ref[0] = jnp.stack(cols, axis=-1)


def _psift_idx(xyz):
    B, N, _ = xyz.shape
    G = 256
    return pl.pallas_call(
        _psift_idx_body,
        grid=(B, N // G),
        in_specs=[
            pl.BlockSpec((1, G, 3), lambda b, i: (b, i, 0)),
            pl.BlockSpec((1, 3, N), lambda b, i: (b, 0, 0)),
        ],
        out_specs=pl.BlockSpec((1, G, 8), lambda b, i: (b, i, 0)),
        out_shape=jax.ShapeDtypeStruct((B, N, 8), jnp.int32),
        interpret=_INTERPRET,
    )(xyz, jnp.transpose(xyz, (0, 2, 1)))


def _bn_nlast(y, g, b):
    red = tuple(range(y.ndim - 1))
    m = jnp.mean(y, axis=red, keepdims=True)
    v = jnp.var(y, axis=red, keepdims=True)
    return (y - m) / jnp.sqrt(v + 1e-5) * g + b


def _pointsift(xyz, feats, layers, radius):
    B, N, _ = xyz.shape
    idx = _psift_idx(xyz)
    bi = jnp.arange(B)[:, None, None]
    gx = xyz[bi, idx] - xyz[:, :, None, :]
    gf = feats[bi, idx]
    x = jnp.concatenate([gx, gf], axis=-1)
    for (W, bb, g, b) in layers:
        Bc, Nn, T, C = x.shape
        xp = x.reshape(Bc, Nn, T // 2, 2, C)
        y = jnp.einsum('bntsc,sco->bnto', xp, W) + bb
        x = jax.nn.relu(_bn_nlast(y, g, b))
    return jnp.transpose(x[:, :, 0, :], (0, 2, 1))


def kernel(template_feature, search_feature, template_xyz, search_xyz,
           template_bc, search_bc, params):
    B, f, M = template_feature.shape
    N = search_feature.shape[2]

    d_bc = _pdist(search_bc, template_bc)
    _, idx_b = jax.lax.top_k(-d_bc, 3 * _K1)
    dxyz = _pdist(search_xyz, template_xyz)
    bi = jnp.arange(B)[:, None, None]
    ni = jnp.arange(N)[None, :, None]
    mask = jnp.full((B, N, M), 1e5, dtype=jnp.float32).at[bi, ni, idx_b].set(1.0)
    _, idx_k = jax.lax.top_k(-(dxyz * mask), _K1)

    s_desc = _scf_desc(search_xyz, _KC)
    t_desc = _scf_desc(template_xyz, _KC)
    cand = t_desc[bi, idx_k]
    dsc = jnp.sqrt(jnp.sum((s_desc[:, :, None, :] - cand) ** 2, axis=-1) + _EPS)
    _, pos4 = jax.lax.top_k(-dsc, _K2)
    idx_scf = jnp.take_along_axis(idx_k, pos4, axis=-1)

    t_oe = _scf_oe_t(template_xyz)
    s_oe = _scf_oe_t(search_xyz)
    t_feat = jnp.concatenate(
        [jnp.transpose(template_feature, (0, 2, 1)), t_oe], axis=-1)
    s_feat = jnp.concatenate(
        [jnp.transpose(search_feature, (0, 2, 1)), s_oe], axis=-1)
    g_feat = t_feat[bi, idx_scf]
    fr = jnp.concatenate([s_feat[:, :, None, :], g_feat], axis=2)

    g_xyz = template_xyz[bi, idx_scf]
    g_bc = template_bc[bi, idx_scf]
    st = jnp.concatenate([
        jnp.concatenate([search_xyz[:, :, None, :], g_xyz], axis=2),
        jnp.concatenate([search_bc[:, :, None, :], g_bc], axis=2),
    ], axis=-1)

    x = fr
    for (W, g, b) in params['mlp']:
        y = jnp.einsum('bnkc,oc->bnko', x, W)
        x = jax.nn.relu(_bn_nlast(y, g, b))

    Bn = B * N
    rel0 = st[:, :, 0:1, :] - st
    ff = _attn0(x.reshape(Bn, _K2 + 1, f),
                rel0.reshape(Bn, _K2 + 1, 12), params['attn'])
    ff = jnp.transpose(ff.reshape(B, N, f), (0, 2, 1))

    ff = _inorm(ff + search_feature)
    y = jnp.einsum('bcn,oc->bon', ff, params['fea_w1'])
    m = jnp.mean(y, axis=(0, 2), keepdims=True)
    v = jnp.var(y, axis=(0, 2), keepdims=True)
    y = jax.nn.relu((y - m) / jnp.sqrt(v + 1e-5)
                    * params['fea_g1'][None, :, None] + params['fea_b1'][None, :, None])
    y = jnp.einsum('bcn,oc->bon', y, params['fea_w2']) + params['fea_bias2'][None, :, None]
    fff = _inorm(y + ff)

    fc = _pointsift(search_xyz, jnp.transpose(fff, (0, 2, 1)), params['oe1'], _RADIUS)
    fc = fc + fff
    fcf = _pointsift(search_xyz, jnp.transpose(fc, (0, 2, 1)), params['oe2'], _RADIUS)
    return fcf + fc


# speedup vs baseline: 1.2165x; 1.0975x over previous
import jax
import jax.numpy as jnp
from jax.experimental import pallas as pl

_EPS = 1e-12
_K1, _K2, _KC, _RADIUS = 8, 4, 16, 0.5


def _pdist(a, b):
    return jnp.sqrt(jnp.sum((a[:, :, None, :] - b[:, None, :, :]) ** 2, axis=-1) + _EPS)


def _scf_oe_t(xyz):
    c = xyz - jnp.mean(xyz, axis=1, keepdims=True)
    r = jnp.sqrt(jnp.sum(c ** 2, axis=-1) + _EPS)
    theta = jnp.arccos(jnp.clip(c[..., 2] / r, -1.0 + 1e-6, 1.0 - 1e-6))
    phi = jnp.arctan2(c[..., 1], c[..., 0] + 1e-8)
    return jnp.stack([r, theta, phi], axis=-1)


def _desc_body(xq_ref, xa_ref, o_ref):
    G = xq_ref.shape[1]
    N = xa_ref.shape[2]
    dx = xa_ref[0, 0:1, :] - xq_ref[0, :, 0:1]
    dy = xa_ref[0, 1:2, :] - xq_ref[0, :, 1:2]
    dz = xa_ref[0, 2:3, :] - xq_ref[0, :, 2:3]
    md = jnp.sqrt(dx * dx + dy * dy + dz * dz + _EPS)
    jj = jax.lax.broadcasted_iota(jnp.int32, (G, N), 1)
    vals = []
    for t in range(_KC + 1):
        mn = jnp.min(md, axis=1)
        j = jnp.argmin(md, axis=1).astype(jnp.int32)
        md = jnp.where(jj == j[:, None], 1e30, md)
        vals.append(mn)
    o_ref[0] = jnp.stack(vals[1:], axis=-1)


def _scf_desc(xyz, kc):
    B, N, _ = xyz.shape
    G = 256
    return pl.pallas_call(
        _desc_body,
        grid=(B, N // G),
        in_specs=[
            pl.BlockSpec((1, G, 3), lambda b, i: (b, i, 0)),
            pl.BlockSpec((1, 3, N), lambda b, i: (b, 0, 0)),
        ],
        out_specs=pl.BlockSpec((1, G, kc), lambda b, i: (b, i, 0)),
        out_shape=jax.ShapeDtypeStruct((B, N, kc), jnp.float32),
    )(xyz, jnp.transpose(xyz, (0, 2, 1)))


def _inorm(x):
    m = jnp.mean(x, axis=1, keepdims=True)
    v = jnp.var(x, axis=1, keepdims=True)
    return (x - m) / jnp.sqrt(v + 1e-5)


def _attn0_body(x_ref, r_ref, wq_ref, wk_ref, wv_ref, pw1_ref, pb1_ref,
                pw2_ref, pb2_ref, aw1_ref, ab1_ref, aw2_ref, ab2_ref, o_ref):
    X = x_ref[...]
    R = r_ref[...]
    G5 = X.shape[0]
    G = G5 // 5
    pe = jax.nn.relu(jnp.dot(R, pw1_ref[...], preferred_element_type=jnp.float32)
                     + pb1_ref[...])
    pe = jnp.dot(pe, pw2_ref[...], preferred_element_type=jnp.float32) + pb2_ref[...]
    q = jnp.dot(X, wq_ref[...], preferred_element_type=jnp.float32)
    k = jnp.dot(X, wk_ref[...], preferred_element_type=jnp.float32)
    v = jnp.dot(X, wv_ref[...], preferred_element_type=jnp.float32)
    pe3 = pe.reshape(G, 5, 128)
    t = q.reshape(G, 5, 128)[:, 0:1, :] - k.reshape(G, 5, 128) + pe3
    h = jax.nn.relu(jnp.dot(t.reshape(G5, 128), aw1_ref[...],
                            preferred_element_type=jnp.float32) + ab1_ref[...])
    h = jnp.dot(h, aw2_ref[...], preferred_element_type=jnp.float32) + ab2_ref[...]
    h3 = h.reshape(G, 5, 128)
    m = jnp.max(h3, axis=1, keepdims=True)
    e = jnp.exp(h3 - m)
    a = e / jnp.sum(e, axis=1, keepdims=True)
    vv = v.reshape(G, 5, 128) + pe3
    o_ref[...] = jnp.sum(a * vv, axis=1)


def _attn0(x, rel0, p):
    Bn = x.shape[0]
    G = 512
    X2 = x.reshape(Bn * 5, 128)
    R2 = rel0.reshape(Bn * 5, 12)
    pb1 = p['pos_b1'].reshape(1, -1)
    pb2 = p['pos_b2'].reshape(1, -1)
    ab1 = p['att_b1'].reshape(1, -1)
    ab2 = p['att_b2'].reshape(1, -1)
    full = lambda a: pl.BlockSpec(a.shape, lambda i: (0, 0))
    return pl.pallas_call(
        _attn0_body,
        grid=(Bn // G,),
        in_specs=[
            pl.BlockSpec((G * 5, 128), lambda i: (i, 0)),
            pl.BlockSpec((G * 5, 12), lambda i: (i, 0)),
            full(p['wq']), full(p['wk']), full(p['wv']),
            full(p['pos_w1']), full(pb1), full(p['pos_w2']), full(pb2),
            full(p['att_w1']), full(ab1), full(p['att_w2']), full(ab2),
        ],
        out_specs=pl.BlockSpec((G, 128), lambda i: (i, 0)),
        out_shape=jax.ShapeDtypeStruct((Bn, 128), jnp.float32),
    )(X2, R2, p['wq'], p['wk'], p['wv'], p['pos_w1'], pb1, p['pos_w2'], pb2,
      p['att_w1'], ab1, p['att_w2'], ab2)


def _psift_idx_body(xq_ref, xa_ref, o_ref):
    G = xq_ref.shape[1]
    N = xa_ref.shape[2]
    i = pl.program_id(1)
    dx = xa_ref[0, 0:1, :] - xq_ref[0, :, 0:1]
    dy = xa_ref[0, 1:2, :] - xq_ref[0, :, 1:2]
    dz = xa_ref[0, 2:3, :] - xq_ref[0, :, 2:3]
    dist = jnp.sqrt(dx * dx + dy * dy + dz * dz + _EPS)
    oc = ((dx >= 0).astype(jnp.int32) * 4
          + (dy >= 0).astype(jnp.int32) * 2
          + (dz >= 0).astype(jnp.int32))
    jj = jax.lax.broadcasted_iota(jnp.int32, (G, N), 1)
    rows = jax.lax.broadcasted_iota(jnp.int32, (G, N), 0) + i * G
    eye = jj == rows
    self_idx = rows[:, 0]
    cols = []
    for o in range(8):
        valid = (oc == o) & (dist <= _RADIUS) & (~eye)
        md = jnp.where(valid, dist, 1e30)
        mn = jnp.min(md, axis=1)
        j = jnp.argmin(md, axis=1).astype(jnp.int32)
        cols.append(jnp.where(mn < 1e30, j, self_idx))
    o_ref[0] = jnp.stack(cols, axis=-1)


def _psift_idx(xyz):
    B, N, _ = xyz.shape
    G = 256
    return pl.pallas_call(
        _psift_idx_body,
        grid=(B, N // G),
        in_specs=[
            pl.BlockSpec((1, G, 3), lambda b, i: (b, i, 0)),
            pl.BlockSpec((1, 3, N), lambda b, i: (b, 0, 0)),
        ],
        out_specs=pl.BlockSpec((1, G, 8), lambda b, i: (b, i, 0)),
        out_shape=jax.ShapeDtypeStruct((B, N, 8), jnp.int32),
    )(xyz, jnp.transpose(xyz, (0, 2, 1)))


def _bn_nlast(y, g, b):
    red = tuple(range(y.ndim - 1))
    m = jnp.mean(y, axis=red, keepdims=True)
    v = jnp.var(y, axis=red, keepdims=True)
    return (y - m) / jnp.sqrt(v + 1e-5) * g + b


def _pointsift(xyz, feats, layers, radius):
    B, N, _ = xyz.shape
    idx = _psift_idx(xyz)
    bi = jnp.arange(B)[:, None, None]
    gx = xyz[bi, idx] - xyz[:, :, None, :]
    gf = feats[bi, idx]
    x = jnp.concatenate([gx, gf], axis=-1)
    for (W, bb, g, b) in layers:
        Bc, Nn, T, C = x.shape
        xp = x.reshape(Bc, Nn, T // 2, 2, C)
        y = jnp.einsum('bntsc,sco->bnto', xp, W) + bb
        x = jax.nn.relu(_bn_nlast(y, g, b))
    return jnp.transpose(x[:, :, 0, :], (0, 2, 1))


def kernel(template_feature, search_feature, template_xyz, search_xyz,
           template_bc, search_bc, params):
    B, f, M = template_feature.shape
    N = search_feature.shape[2]

    d_bc = _pdist(search_bc, template_bc)
    _, idx_b = jax.lax.top_k(-d_bc, 3 * _K1)
    dxyz = _pdist(search_xyz, template_xyz)
    bi = jnp.arange(B)[:, None, None]
    ni = jnp.arange(N)[None, :, None]
    mask = jnp.full((B, N, M), 1e5, dtype=jnp.float32).at[bi, ni, idx_b].set(1.0)
    _, idx_k = jax.lax.top_k(-(dxyz * mask), _K1)

    s_desc = _scf_desc(search_xyz, _KC)
    t_desc = _scf_desc(template_xyz, _KC)
    cand = t_desc[bi, idx_k]
    dsc = jnp.sqrt(jnp.sum((s_desc[:, :, None, :] - cand) ** 2, axis=-1) + _EPS)
    _, pos4 = jax.lax.top_k(-dsc, _K2)
    idx_scf = jnp.take_along_axis(idx_k, pos4, axis=-1)

    t_oe = _scf_oe_t(template_xyz)
    s_oe = _scf_oe_t(search_xyz)
    t_feat = jnp.concatenate(
        [jnp.transpose(template_feature, (0, 2, 1)), t_oe], axis=-1)
    s_feat = jnp.concatenate(
        [jnp.transpose(search_feature, (0, 2, 1)), s_oe], axis=-1)
    g_feat = t_feat[bi, idx_scf]
    fr = jnp.concatenate([s_feat[:, :, None, :], g_feat], axis=2)

    g_xyz = template_xyz[bi, idx_scf]
    g_bc = template_bc[bi, idx_scf]
    st = jnp.concatenate([
        jnp.concatenate([search_xyz[:, :, None, :], g_xyz], axis=2),
        jnp.concatenate([search_bc[:, :, None, :], g_bc], axis=2),
    ], axis=-1)

    x = fr
    for (W, g, b) in params['mlp']:
        y = jnp.einsum('bnkc,oc->bnko', x, W)
        x = jax.nn.relu(_bn_nlast(y, g, b))

    Bn = B * N
    rel0 = st[:, :, 0:1, :] - st
    ff = _attn0(x.reshape(Bn, _K2 + 1, f),
                rel0.reshape(Bn, _K2 + 1, 12), params['attn'])
    ff = jnp.transpose(ff.reshape(B, N, f), (0, 2, 1))

    ff = _inorm(ff + search_feature)
    y = jnp.einsum('bcn,oc->bon', ff, params['fea_w1'])
    m = jnp.mean(y, axis=(0, 2), keepdims=True)
    v = jnp.var(y, axis=(0, 2), keepdims=True)
    y = jax.nn.relu((y - m) / jnp.sqrt(v + 1e-5)
                    * params['fea_g1'][None, :, None] + params['fea_b1'][None, :, None])
    y = jnp.einsum('bcn,oc->bon', y, params['fea_w2']) + params['fea_bias2'][None, :, None]
    fff = _inorm(y + ff)

    fc = _pointsift(search_xyz, jnp.transpose(fff, (0, 2, 1)), params['oe1'], _RADIUS)
    fc = fc + fff
    fcf = _pointsift(search_xyz, jnp.transpose(fc, (0, 2, 1)), params['oe2'], _RADIUS)
    return fcf + fc
